# initial kernel scaffold (unmeasured)
import jax
import jax.numpy as jnp
from jax import lax
from jax.experimental import pallas as pl
from jax.experimental.pallas import tpu as pltpu

N_DEV = 4
N_STEPS = 2 * (N_DEV - 1)

_sem_signal = getattr(pl, "semaphore_signal", None) or pltpu.semaphore_signal
_sem_wait = getattr(pl, "semaphore_wait", None) or pltpu.semaphore_wait


def kernel(dy, W):
    partial = lax.dot_general(
        dy.astype(jnp.bfloat16),
        W.astype(jnp.bfloat16),
        dimension_numbers=(((1,), (1,)), ((), ())),
        preferred_element_type=jnp.float32,
    )
    m, n = partial.shape
    chunk = m // N_DEV

    def body(x_ref, out_ref, send_buf, recv_buf, send_sems, recv_sems):
        my_x = lax.axis_index("x")
        my_y = lax.axis_index("y")
        my_z = lax.axis_index("z")
        right = (my_z + 1) % N_DEV
        left = (my_z + N_DEV - 1) % N_DEV

        barrier_sem = pltpu.get_barrier_semaphore()
        for nbr in (left, right):
            _sem_signal(
                barrier_sem,
                inc=1,
                device_id=(my_x, my_y, nbr),
                device_id_type=pl.DeviceIdType.MESH,
            )
        _sem_wait(barrier_sem, 2)

        out_ref[...] = x_ref[...]

        for s in range(N_DEV - 1):
            send_c = (my_z + (N_DEV - s)) % N_DEV
            recv_c = (my_z + (N_DEV - s - 1)) % N_DEV
            send_buf[s] = out_ref[pl.ds(send_c * chunk, chunk), :].astype(
                jnp.bfloat16
            )
            rdma = pltpu.make_async_remote_copy(
                src_ref=send_buf.at[s],
                dst_ref=recv_buf.at[s],
                send_sem=send_sems.at[s],
                recv_sem=recv_sems.at[s],
                device_id=(my_x, my_y, right),
                device_id_type=pl.DeviceIdType.MESH,
            )
            rdma.start()
            rdma.wait()
            out_ref[pl.ds(recv_c * chunk, chunk), :] = (
                out_ref[pl.ds(recv_c * chunk, chunk), :]
                + recv_buf[s].astype(jnp.float32)
            )

        for s in range(N_DEV - 1):
            j = (N_DEV - 1) + s
            send_c = (my_z + (N_DEV + 1 - s)) % N_DEV
            recv_c = (my_z + (N_DEV - s)) % N_DEV
            send_buf[j] = out_ref[pl.ds(send_c * chunk, chunk), :].astype(
                jnp.bfloat16
            )
            rdma = pltpu.make_async_remote_copy(
                src_ref=send_buf.at[j],
                dst_ref=recv_buf.at[j],
                send_sem=send_sems.at[j],
                recv_sem=recv_sems.at[j],
                device_id=(my_x, my_y, right),
                device_id_type=pl.DeviceIdType.MESH,
            )
            rdma.start()
            rdma.wait()
            out_ref[pl.ds(recv_c * chunk, chunk), :] = recv_buf[j].astype(
                jnp.float32
            )

    return pl.pallas_call(
        body,
        out_shape=jax.ShapeDtypeStruct((m, n), jnp.float32),
        in_specs=[pl.BlockSpec(memory_space=pltpu.VMEM)],
        out_specs=pl.BlockSpec(memory_space=pltpu.VMEM),
        scratch_shapes=[
            pltpu.VMEM((N_STEPS, chunk, n), jnp.bfloat16),
            pltpu.VMEM((N_STEPS, chunk, n), jnp.bfloat16),
            pltpu.SemaphoreType.DMA((N_STEPS,)),
            pltpu.SemaphoreType.DMA((N_STEPS,)),
        ],
        compiler_params=pltpu.CompilerParams(collective_id=0),
    )(partial)


# baseline (device time: 282724 ns/iter reference)
import jax
import jax.numpy as jnp
from jax import lax
from jax.experimental import pallas as pl
from jax.experimental.pallas import tpu as pltpu

N_DEV = 4
N_STEPS = 2 * (N_DEV - 1)

_sem_signal = getattr(pl, "semaphore_signal", None) or pltpu.semaphore_signal
_sem_wait = getattr(pl, "semaphore_wait", None) or pltpu.semaphore_wait


def kernel(dy, W):
    partial = lax.dot_general(
        dy.astype(jnp.bfloat16),
        W.astype(jnp.bfloat16),
        dimension_numbers=(((1,), (1,)), ((), ())),
        preferred_element_type=jnp.float32,
    )
    m, n = partial.shape
    chunk = m // N_DEV

    def body(x_ref, out_ref, send_buf, recv_buf, send_sems, recv_sems):
        my_x = lax.axis_index("x")
        my_y = lax.axis_index("y")
        my_z = lax.axis_index("z")
        right = (my_z + 1) % N_DEV
        left = (my_z + N_DEV - 1) % N_DEV

        barrier_sem = pltpu.get_barrier_semaphore()
        for nbr in (left, right):
            _sem_signal(
                barrier_sem,
                inc=1,
                device_id=(my_x, my_y, nbr),
                device_id_type=pl.DeviceIdType.MESH,
            )
        _sem_wait(barrier_sem, 2)

        out_ref[...] = x_ref[...]

        for s in range(N_DEV - 1):
            send_c = (my_z + (N_DEV - s)) % N_DEV
            recv_c = (my_z + (N_DEV - s - 1)) % N_DEV
            send_buf[s] = out_ref[pl.ds(send_c * chunk, chunk), :].astype(
                jnp.bfloat16
            )
            rdma = pltpu.make_async_remote_copy(
                src_ref=send_buf.at[s],
                dst_ref=recv_buf.at[s],
                send_sem=send_sems.at[s],
                recv_sem=recv_sems.at[s],
                device_id=(my_x, my_y, right),
                device_id_type=pl.DeviceIdType.MESH,
            )
            rdma.start()
            rdma.wait()
            out_ref[pl.ds(recv_c * chunk, chunk), :] = (
                out_ref[pl.ds(recv_c * chunk, chunk), :]
                + recv_buf[s].astype(jnp.float32)
            )

        for s in range(N_DEV - 1):
            j = (N_DEV - 1) + s
            send_c = (my_z + (N_DEV + 1 - s)) % N_DEV
            recv_c = (my_z + (N_DEV - s)) % N_DEV
            send_buf[j] = out_ref[pl.ds(send_c * chunk, chunk), :].astype(
                jnp.bfloat16
            )
            rdma = pltpu.make_async_remote_copy(
                src_ref=send_buf.at[j],
                dst_ref=recv_buf.at[j],
                send_sem=send_sems.at[j],
                recv_sem=recv_sems.at[j],
                device_id=(my_x, my_y, right),
                device_id_type=pl.DeviceIdType.MESH,
            )
            rdma.start()
            rdma.wait()
            out_ref[pl.ds(recv_c * chunk, chunk), :] = recv_buf[j].astype(
                jnp.float32
            )

    return pl.pallas_call(
        body,
        out_shape=jax.ShapeDtypeStruct((m, n), jnp.float32),
        in_specs=[pl.BlockSpec(memory_space=pltpu.VMEM)],
        out_specs=pl.BlockSpec(memory_space=pltpu.VMEM),
        scratch_shapes=[
            pltpu.VMEM((N_STEPS, chunk, n), jnp.bfloat16),
            pltpu.VMEM((N_STEPS, chunk, n), jnp.bfloat16),
            pltpu.SemaphoreType.DMA((N_STEPS,)),
            pltpu.SemaphoreType.DMA((N_STEPS,)),
        ],
        compiler_params=pltpu.CompilerParams(
            collective_id=0, vmem_limit_bytes=100 * 1024 * 1024
        ),
    )(partial)
